# baseline (speedup 1.0000x reference)
import functools

import jax
import jax.numpy as jnp
from jax import lax
from jax.experimental import pallas as pl
from jax.experimental.pallas import tpu as pltpu
from jax.experimental.pallas import tpu_sc as plsc

N_NODES = 10000
E_TOTAL = 160000
NC = 2
NS = 16
L = 16
D = 256
NR = 10240
RNG = NR // (NC * NS)
MC = 3200
NM = E_TOTAL // MC
GC = 80

_MESH = plsc.VectorSubcoreMesh(
    core_axis_name="c", subcore_axis_name="s", num_cores=NC, num_subcores=NS
)


def _sc_aggregate(table, srcs, dsts):
    @functools.partial(
        pl.kernel,
        out_type=(
            jax.ShapeDtypeStruct((NR, D), jnp.float32),
            jax.ShapeDtypeStruct((NR,), jnp.float32),
        ),
        mesh=_MESH,
        compiler_params=pltpu.CompilerParams(needs_layout_passes=False),
        scratch_types=[
            pltpu.VMEM((MC,), jnp.int32),
            pltpu.VMEM((MC,), jnp.int32),
            pltpu.VMEM((MC,), jnp.int32),
            pltpu.VMEM((MC,), jnp.int32),
            pltpu.VMEM((MC,), jnp.int32),
            pltpu.VMEM((MC,), jnp.int32),
            pltpu.VMEM((GC, D), jnp.float32),
            pltpu.VMEM((RNG, D), jnp.float32),
            pltpu.VMEM((RNG,), jnp.float32),
            pltpu.SemaphoreType.DMA,
            pltpu.SemaphoreType.DMA,
            pltpu.SemaphoreType.DMA,
        ],
    )
    def agg(table_h, src_h, dst_h, sum_o, cnt_o,
            smac0, dmac0, smac1, dmac1, glist, dlist, rows, acc, lcnt,
            sem, sem0, sem1):
        c = lax.axis_index("c")
        s = lax.axis_index("s")
        wid = c * NS + s
        base = wid * RNG
        zeros16 = jnp.zeros((L,), jnp.float32)
        ones16 = jnp.ones((L,), jnp.float32)
        lanes = lax.iota(jnp.int32, L)
        lane0 = lanes == 0

        def _zacc(r, carry):
            def _zc(j, cc):
                acc[r, pl.ds(j * L, L)] = zeros16
                return cc
            return lax.fori_loop(0, D // L, _zc, carry)
        lax.fori_loop(0, RNG, _zacc, 0)

        def _zcnt(i, carry):
            lcnt[pl.ds(i * L, L)] = zeros16
            return carry
        lax.fori_loop(0, RNG // L, _zcnt, 0)

        zi16 = jnp.zeros((L,), jnp.int32)

        def _zg(i, carry):
            glist[pl.ds(i * L, L)] = zi16
            return carry
        lax.fori_loop(0, MC // L, _zg, 0)

        def _edge(dvec, ii, rp):
            dl_s = dvec[ii]
            rowv = jnp.full((L,), dl_s, jnp.int32)
            for jj in range(D // L):
                plsc.addupdate_scatter(acc, [rowv, lanes + jj * L],
                                       rows[rp, pl.ds(jj * L, L)])
            plsc.addupdate_scatter(lcnt, [rowv], ones16, mask=lane0)

        def _process(smac, dmac, m):
            mb = m * MC

            def _fchunk(j, off):
                d = dmac[pl.ds(j * L, L)]
                sv = smac[pl.ds(j * L, L)]
                dl = d - base
                owned = jnp.logical_and(dl >= 0, dl < RNG)
                incl = plsc.cumsum(owned.astype(jnp.int32))
                pos = off + incl - 1
                plsc.store_scatter(glist, [pos], sv, mask=owned)
                plsc.store_scatter(dlist, [pos], dl, mask=owned)
                return off + incl[L - 1]
            nown = lax.fori_loop(0, MC // L, _fchunk, jnp.int32(0))

            nch = (nown + GC - 1) // GC

            def _chunk(cidx, carry):
                cb = cidx * GC
                pltpu.async_copy(table_h.at[glist.at[pl.ds(cb, GC)]],
                                 rows, sem).wait()
                ne = jnp.minimum(GC, nown - cb)
                nfull = ne // L

                def _sub(k, cc):
                    dvec = dlist[pl.ds(cb + k * L, L)]
                    for ii in range(L):
                        _edge(dvec, ii, k * L + ii)
                    return cc
                lax.fori_loop(0, nfull, _sub, 0)

                @pl.when(nfull * L < ne)
                def _():
                    dvec = dlist[pl.ds(cb + nfull * L, L)]
                    for ii in range(L):
                        @pl.when(nfull * L + ii < ne)
                        def _():
                            _edge(dvec, ii, nfull * L + ii)
                return carry
            lax.fori_loop(0, nch, _chunk, 0)

        def _issue(m, smac, dmac, semx):
            mb = m * MC
            a = pltpu.async_copy(src_h.at[pl.ds(mb, MC)], smac, semx)
            b = pltpu.async_copy(dst_h.at[pl.ds(mb, MC)], dmac, semx)
            return a, b

        def _wait(smac, dmac, semx):
            pltpu.make_async_copy(src_h.at[pl.ds(0, MC)], smac, semx).wait()
            pltpu.make_async_copy(dst_h.at[pl.ds(0, MC)], dmac, semx).wait()

        _issue(0, smac0, dmac0, sem0)

        def _pair(t, carry):
            _wait(smac0, dmac0, sem0)
            _issue(2 * t + 1, smac1, dmac1, sem1)
            _process(smac0, dmac0, 2 * t)
            _wait(smac1, dmac1, sem1)

            @pl.when(t < NM // 2 - 1)
            def _():
                _issue(2 * t + 2, smac0, dmac0, sem0)
            _process(smac1, dmac1, 2 * t + 1)
            return carry
        lax.fori_loop(0, NM // 2, _pair, 0)

        pltpu.sync_copy(acc, sum_o.at[pl.ds(base, RNG)])
        pltpu.sync_copy(lcnt, cnt_o.at[pl.ds(base, RNG)])

    return agg(table, srcs, dsts)


def _dot(a, b):
    return lax.dot_general(a, b, (((1,), (0,)), ((), ())),
                           preferred_element_type=jnp.float32)


def _tc_dense0(s0, c0, x, W_l, W_r, b):
    M = 1000
    H = W_l.shape[1]

    def body(s_r, c_r, x_r, wl_r, wr_r, b_r, o_r):
        mean = s_r[...] / jnp.maximum(c_r[...], 1.0)
        acc = _dot(mean, wl_r[...]) + _dot(x_r[...], wr_r[...]) + b_r[...]
        o_r[...] = jnp.maximum(acc, 0.0)

    return pl.pallas_call(
        body,
        grid=(s0.shape[0] // M,),
        in_specs=[
            pl.BlockSpec((M, D), lambda i: (i, 0)),
            pl.BlockSpec((M, 1), lambda i: (i, 0)),
            pl.BlockSpec((M, D), lambda i: (i, 0)),
            pl.BlockSpec(W_l.shape, lambda i: (0, 0)),
            pl.BlockSpec(W_r.shape, lambda i: (0, 0)),
            pl.BlockSpec((1, H), lambda i: (0, 0)),
        ],
        out_specs=pl.BlockSpec((M, H), lambda i: (i, 0)),
        out_shape=jax.ShapeDtypeStruct((s0.shape[0], H), jnp.float32),
    )(s0, c0, x, W_l, W_r, b)


def _tc_dense1(s_lo, s_hi, c1, x, Wl_lo, Wl_hi, W_r, b, Wp, bp):
    M = 1000
    H = Wl_lo.shape[1]
    OW = Wp.shape[1]

    def body(lo_r, hi_r, c_r, x_r, wlo_r, whi_r, wr_r, b_r, wp_r, bp_r, o_r):
        inv = 1.0 / jnp.maximum(c_r[...], 1.0)
        acc = (_dot(lo_r[...] * inv, wlo_r[...])
               + _dot(hi_r[...] * inv, whi_r[...])
               + _dot(x_r[...], wr_r[...]) + b_r[...])
        h = jnp.maximum(acc, 0.0)
        o_r[...] = _dot(h, wp_r[...]) + bp_r[...]

    return pl.pallas_call(
        body,
        grid=(s_lo.shape[0] // M,),
        in_specs=[
            pl.BlockSpec((M, D), lambda i: (i, 0)),
            pl.BlockSpec((M, D), lambda i: (i, 0)),
            pl.BlockSpec((M, 1), lambda i: (i, 0)),
            pl.BlockSpec((M, D), lambda i: (i, 0)),
            pl.BlockSpec(Wl_lo.shape, lambda i: (0, 0)),
            pl.BlockSpec(Wl_hi.shape, lambda i: (0, 0)),
            pl.BlockSpec(W_r.shape, lambda i: (0, 0)),
            pl.BlockSpec((1, H), lambda i: (0, 0)),
            pl.BlockSpec(Wp.shape, lambda i: (0, 0)),
            pl.BlockSpec((1, OW), lambda i: (0, 0)),
        ],
        out_specs=pl.BlockSpec((M, OW), lambda i: (i, 0)),
        out_shape=jax.ShapeDtypeStruct((s_lo.shape[0], OW), jnp.float32),
    )(s_lo, s_hi, c1, x, Wl_lo, Wl_hi, W_r, b, Wp, bp)


def kernel(x_paper, x_author, edge_index_p2a, edge_index_a2p,
           W0_l, b0, W0_r, W1_l, b1, W1_r, Wp, bp):
    src0 = edge_index_p2a[0].astype(jnp.int32)
    dst0 = edge_index_p2a[1].astype(jnp.int32)
    src1 = edge_index_a2p[0].astype(jnp.int32)
    dst1 = edge_index_a2p[1].astype(jnp.int32)

    s0, c0 = _sc_aggregate(x_paper, src0, dst0)
    h_author = _tc_dense0(s0[:N_NODES], c0[:N_NODES, None], x_author,
                          W0_l, W0_r, b0[None, :])
    s1lo, c1 = _sc_aggregate(h_author[:, :D], src1, dst1)
    s1hi, _ = _sc_aggregate(h_author[:, D:], src1, dst1)
    out = _tc_dense1(s1lo[:N_NODES], s1hi[:N_NODES], c1[:N_NODES, None],
                     x_paper, W1_l[:D], W1_l[D:], W1_r, b1[None, :],
                     Wp, bp[None, :])
    return out


# speedup vs baseline: 1.0013x; 1.0013x over previous
import functools

import jax
import jax.numpy as jnp
from jax import lax
from jax.experimental import pallas as pl
from jax.experimental.pallas import tpu as pltpu
from jax.experimental.pallas import tpu_sc as plsc

N_NODES = 10000
E_TOTAL = 160000
NC = 2
NS = 16
L = 16
D = 256
NR = 10240
RNG = NR // (NC * NS)
MC = 3200
NM = E_TOTAL // MC
GC = 80

_MESH = plsc.VectorSubcoreMesh(
    core_axis_name="c", subcore_axis_name="s", num_cores=NC, num_subcores=NS
)


def _sc_aggregate(table, srcs, dsts):
    @functools.partial(
        pl.kernel,
        out_type=(
            jax.ShapeDtypeStruct((NR, D), jnp.float32),
            jax.ShapeDtypeStruct((NR,), jnp.float32),
        ),
        mesh=_MESH,
        compiler_params=pltpu.CompilerParams(needs_layout_passes=False),
        scratch_types=[
            pltpu.VMEM((MC,), jnp.int32),
            pltpu.VMEM((MC,), jnp.int32),
            pltpu.VMEM((MC,), jnp.int32),
            pltpu.VMEM((MC,), jnp.int32),
            pltpu.VMEM((MC,), jnp.int32),
            pltpu.VMEM((MC,), jnp.int32),
            pltpu.VMEM((GC, D), jnp.float32),
            pltpu.VMEM((RNG, D), jnp.float32),
            pltpu.VMEM((RNG,), jnp.float32),
            pltpu.SemaphoreType.DMA,
            pltpu.SemaphoreType.DMA,
            pltpu.SemaphoreType.DMA,
        ],
    )
    def agg(table_h, src_h, dst_h, sum_o, cnt_o,
            smac0, dmac0, smac1, dmac1, glist, dlist, rows, acc, lcnt,
            sem, sem0, sem1):
        c = lax.axis_index("c")
        s = lax.axis_index("s")
        wid = c * NS + s
        base = wid * RNG
        zeros16 = jnp.zeros((L,), jnp.float32)
        ones16 = jnp.ones((L,), jnp.float32)
        lanes = lax.iota(jnp.int32, L)
        lane0 = lanes == 0

        def _zacc(r, carry):
            def _zc(j, cc):
                acc[r, pl.ds(j * L, L)] = zeros16
                return cc
            return lax.fori_loop(0, D // L, _zc, carry)
        lax.fori_loop(0, RNG, _zacc, 0)

        def _zcnt(i, carry):
            lcnt[pl.ds(i * L, L)] = zeros16
            return carry
        lax.fori_loop(0, RNG // L, _zcnt, 0)

        zi16 = jnp.zeros((L,), jnp.int32)

        def _zg(i, carry):
            glist[pl.ds(i * L, L)] = zi16
            return carry
        lax.fori_loop(0, MC // L, _zg, 0)

        def _edge(dvec, ii, rp):
            dl_s = dvec[ii]
            rowv = jnp.full((L,), dl_s, jnp.int32)
            for jj in range(D // L):
                plsc.addupdate_scatter(acc, [rowv, lanes + jj * L],
                                       rows[rp, pl.ds(jj * L, L)])
            plsc.addupdate_scatter(lcnt, [rowv], ones16, mask=lane0)

        def _process(smac, dmac, m):
            mb = m * MC

            def _fchunk(j, off):
                d = dmac[pl.ds(j * L, L)]
                sv = smac[pl.ds(j * L, L)]
                dl = d - base
                owned = jnp.logical_and(dl >= 0, dl < RNG)
                incl = plsc.cumsum(owned.astype(jnp.int32))
                pos = off + incl - 1
                plsc.store_scatter(glist, [pos], sv, mask=owned)
                plsc.store_scatter(dlist, [pos], dl, mask=owned)
                return off + incl[L - 1]
            nown = lax.fori_loop(0, MC // L, _fchunk, jnp.int32(0))

            nch = (nown + GC - 1) // GC

            def _chunk(cidx, carry):
                cb = cidx * GC
                pltpu.async_copy(table_h.at[glist.at[pl.ds(cb, GC)]],
                                 rows, sem).wait()
                ne = jnp.minimum(GC, nown - cb)
                nfull = ne // L

                def _sub(k, cc):
                    dvec = dlist[pl.ds(cb + k * L, L)]
                    for ii in range(L):
                        _edge(dvec, ii, k * L + ii)
                    return cc
                lax.fori_loop(0, nfull, _sub, 0)

                @pl.when(nfull * L < ne)
                def _():
                    dvec = dlist[pl.ds(cb + nfull * L, L)]
                    for ii in range(L):
                        @pl.when(nfull * L + ii < ne)
                        def _():
                            _edge(dvec, ii, nfull * L + ii)
                return carry
            lax.fori_loop(0, nch, _chunk, 0)

        def _macro(m, carry):
            mb = m * MC
            pltpu.sync_copy(src_h.at[pl.ds(mb, MC)], smac0)
            pltpu.sync_copy(dst_h.at[pl.ds(mb, MC)], dmac0)
            _process(smac0, dmac0, m)
            return carry
        lax.fori_loop(0, NM, _macro, 0)

        pltpu.sync_copy(acc, sum_o.at[pl.ds(base, RNG)])
        pltpu.sync_copy(lcnt, cnt_o.at[pl.ds(base, RNG)])

    return agg(table, srcs, dsts)


def _dot(a, b):
    return lax.dot_general(a, b, (((1,), (0,)), ((), ())),
                           preferred_element_type=jnp.float32)


def _tc_dense0(s0, c0, x, W_l, W_r, b):
    M = 1000
    H = W_l.shape[1]

    def body(s_r, c_r, x_r, wl_r, wr_r, b_r, o_r):
        mean = s_r[...] / jnp.maximum(c_r[...], 1.0)
        acc = _dot(mean, wl_r[...]) + _dot(x_r[...], wr_r[...]) + b_r[...]
        o_r[...] = jnp.maximum(acc, 0.0)

    return pl.pallas_call(
        body,
        grid=(s0.shape[0] // M,),
        in_specs=[
            pl.BlockSpec((M, D), lambda i: (i, 0)),
            pl.BlockSpec((M, 1), lambda i: (i, 0)),
            pl.BlockSpec((M, D), lambda i: (i, 0)),
            pl.BlockSpec(W_l.shape, lambda i: (0, 0)),
            pl.BlockSpec(W_r.shape, lambda i: (0, 0)),
            pl.BlockSpec((1, H), lambda i: (0, 0)),
        ],
        out_specs=pl.BlockSpec((M, H), lambda i: (i, 0)),
        out_shape=jax.ShapeDtypeStruct((s0.shape[0], H), jnp.float32),
    )(s0, c0, x, W_l, W_r, b)


def _tc_dense1(s_lo, s_hi, c1, x, Wl_lo, Wl_hi, W_r, b, Wp, bp):
    M = 1000
    H = Wl_lo.shape[1]
    OW = Wp.shape[1]

    def body(lo_r, hi_r, c_r, x_r, wlo_r, whi_r, wr_r, b_r, wp_r, bp_r, o_r):
        inv = 1.0 / jnp.maximum(c_r[...], 1.0)
        acc = (_dot(lo_r[...] * inv, wlo_r[...])
               + _dot(hi_r[...] * inv, whi_r[...])
               + _dot(x_r[...], wr_r[...]) + b_r[...])
        h = jnp.maximum(acc, 0.0)
        o_r[...] = _dot(h, wp_r[...]) + bp_r[...]

    return pl.pallas_call(
        body,
        grid=(s_lo.shape[0] // M,),
        in_specs=[
            pl.BlockSpec((M, D), lambda i: (i, 0)),
            pl.BlockSpec((M, D), lambda i: (i, 0)),
            pl.BlockSpec((M, 1), lambda i: (i, 0)),
            pl.BlockSpec((M, D), lambda i: (i, 0)),
            pl.BlockSpec(Wl_lo.shape, lambda i: (0, 0)),
            pl.BlockSpec(Wl_hi.shape, lambda i: (0, 0)),
            pl.BlockSpec(W_r.shape, lambda i: (0, 0)),
            pl.BlockSpec((1, H), lambda i: (0, 0)),
            pl.BlockSpec(Wp.shape, lambda i: (0, 0)),
            pl.BlockSpec((1, OW), lambda i: (0, 0)),
        ],
        out_specs=pl.BlockSpec((M, OW), lambda i: (i, 0)),
        out_shape=jax.ShapeDtypeStruct((s_lo.shape[0], OW), jnp.float32),
    )(s_lo, s_hi, c1, x, Wl_lo, Wl_hi, W_r, b, Wp, bp)


def kernel(x_paper, x_author, edge_index_p2a, edge_index_a2p,
           W0_l, b0, W0_r, W1_l, b1, W1_r, Wp, bp):
    src0 = edge_index_p2a[0].astype(jnp.int32)
    dst0 = edge_index_p2a[1].astype(jnp.int32)
    src1 = edge_index_a2p[0].astype(jnp.int32)
    dst1 = edge_index_a2p[1].astype(jnp.int32)

    s0, c0 = _sc_aggregate(x_paper, src0, dst0)
    h_author = _tc_dense0(s0[:N_NODES], c0[:N_NODES, None], x_author,
                          W0_l, W0_r, b0[None, :])
    s1lo, c1 = _sc_aggregate(h_author[:, :D], src1, dst1)
    s1hi, _ = _sc_aggregate(h_author[:, D:], src1, dst1)
    out = _tc_dense1(s1lo[:N_NODES], s1hi[:N_NODES], c1[:N_NODES, None],
                     x_paper, W1_l[:D], W1_l[D:], W1_r, b1[None, :],
                     Wp, bp[None, :])
    return out


# speedup vs baseline: 2.7549x; 2.7514x over previous
import functools

import jax
import jax.numpy as jnp
from jax import lax
from jax.experimental import pallas as pl
from jax.experimental.pallas import tpu as pltpu
from jax.experimental.pallas import tpu_sc as plsc

N_NODES = 10000
E_TOTAL = 160000
NC = 2
NS = 16
L = 16
D = 256
NR = 10240
RNG = NR // (NC * NS)
MC = 1600
NM = E_TOTAL // MC
GC = 64

_MESH = plsc.VectorSubcoreMesh(
    core_axis_name="c", subcore_axis_name="s", num_cores=NC, num_subcores=NS
)


def _sc_aggregate(table, srcs, dsts):
    @functools.partial(
        pl.kernel,
        out_type=(
            jax.ShapeDtypeStruct((NR, D), jnp.float32),
            jax.ShapeDtypeStruct((NR,), jnp.float32),
        ),
        mesh=_MESH,
        compiler_params=pltpu.CompilerParams(needs_layout_passes=False),
        scratch_types=[
            pltpu.VMEM((MC,), jnp.int32),
            pltpu.VMEM((MC,), jnp.int32),
            pltpu.VMEM((MC,), jnp.int32),
            pltpu.VMEM((MC,), jnp.int32),
            pltpu.VMEM((MC,), jnp.int32),
            pltpu.VMEM((MC,), jnp.int32),
            pltpu.VMEM((GC, D), jnp.float32),
            pltpu.VMEM((RNG, D), jnp.float32),
            pltpu.VMEM((RNG,), jnp.float32),
            pltpu.SemaphoreType.DMA,
            pltpu.SemaphoreType.DMA,
            pltpu.SemaphoreType.DMA,
        ],
    )
    def agg(table_h, src_h, dst_h, sum_o, cnt_o,
            smac0, dmac0, smac1, dmac1, glist, dlist, rows, acc, lcnt,
            sem, sem0, sem1):
        c = lax.axis_index("c")
        s = lax.axis_index("s")
        wid = c * NS + s
        base = wid * RNG
        zeros16 = jnp.zeros((L,), jnp.float32)
        ones16 = jnp.ones((L,), jnp.float32)
        lanes = lax.iota(jnp.int32, L)
        lane0 = lanes == 0

        def _zacc(r, carry):
            def _zc(j, cc):
                acc[r, pl.ds(j * L, L)] = zeros16
                return cc
            return lax.fori_loop(0, D // L, _zc, carry)
        lax.fori_loop(0, RNG, _zacc, 0)

        def _zcnt(i, carry):
            lcnt[pl.ds(i * L, L)] = zeros16
            return carry
        lax.fori_loop(0, RNG // L, _zcnt, 0)

        zi16 = jnp.zeros((L,), jnp.int32)

        def _zg(i, carry):
            glist[pl.ds(i * L, L)] = zi16
            return carry
        lax.fori_loop(0, MC // L, _zg, 0)

        def _edge(dvec, ii, rp):
            dl_s = dvec[ii]
            rowv = jnp.full((L,), dl_s, jnp.int32)
            for jj in range(D // L):
                plsc.addupdate_scatter(acc, [rowv, lanes + jj * L],
                                       rows[rp, pl.ds(jj * L, L)])
            plsc.addupdate_scatter(lcnt, [rowv], ones16, mask=lane0)

        def _process(smac, dmac, m):
            mb = m * MC

            def _fchunk(j, off):
                d = dmac[pl.ds(j * L, L)]
                sv = smac[pl.ds(j * L, L)]
                dl = d - base
                owned = jnp.logical_and(dl >= 0, dl < RNG)
                incl = plsc.cumsum(owned.astype(jnp.int32))
                pos = off + incl - 1
                plsc.store_scatter(glist, [pos], sv, mask=owned)
                plsc.store_scatter(dlist, [pos], dl, mask=owned)
                return off + incl[L - 1]
            nown = lax.fori_loop(0, MC // L, _fchunk, jnp.int32(0))

            nch = (nown + GC - 1) // GC

            def _chunk(cidx, carry):
                cb = cidx * GC
                pltpu.async_copy(table_h.at[glist.at[pl.ds(cb, GC)]],
                                 rows, sem).wait()
                ne = jnp.minimum(GC, nown - cb)

                def _sub(k, cc):
                    dvec = dlist[pl.ds(cb + k * L, L)]
                    for ii in range(L):
                        @pl.when(k * L + ii < ne)
                        def _():
                            _edge(dvec, ii, k * L + ii)
                    return cc
                lax.fori_loop(0, (ne + L - 1) // L, _sub, 0)
                return carry
            lax.fori_loop(0, nch, _chunk, 0)

        def _macro(m, carry):
            mb = m * MC
            pltpu.sync_copy(src_h.at[pl.ds(mb, MC)], smac0)
            pltpu.sync_copy(dst_h.at[pl.ds(mb, MC)], dmac0)
            _process(smac0, dmac0, m)
            return carry
        lax.fori_loop(0, NM, _macro, 0)

        pltpu.sync_copy(acc, sum_o.at[pl.ds(base, RNG)])
        pltpu.sync_copy(lcnt, cnt_o.at[pl.ds(base, RNG)])

    return agg(table, srcs, dsts)


def _dot(a, b):
    return lax.dot_general(a, b, (((1,), (0,)), ((), ())),
                           preferred_element_type=jnp.float32)


def _tc_dense0(s0, c0, x, W_l, W_r, b):
    M = 1000
    H = W_l.shape[1]

    def body(s_r, c_r, x_r, wl_r, wr_r, b_r, o_r):
        mean = s_r[...] / jnp.maximum(c_r[...], 1.0)
        acc = _dot(mean, wl_r[...]) + _dot(x_r[...], wr_r[...]) + b_r[...]
        o_r[...] = jnp.maximum(acc, 0.0)

    return pl.pallas_call(
        body,
        grid=(s0.shape[0] // M,),
        in_specs=[
            pl.BlockSpec((M, D), lambda i: (i, 0)),
            pl.BlockSpec((M, 1), lambda i: (i, 0)),
            pl.BlockSpec((M, D), lambda i: (i, 0)),
            pl.BlockSpec(W_l.shape, lambda i: (0, 0)),
            pl.BlockSpec(W_r.shape, lambda i: (0, 0)),
            pl.BlockSpec((1, H), lambda i: (0, 0)),
        ],
        out_specs=pl.BlockSpec((M, H), lambda i: (i, 0)),
        out_shape=jax.ShapeDtypeStruct((s0.shape[0], H), jnp.float32),
    )(s0, c0, x, W_l, W_r, b)


def _tc_dense1(s_lo, s_hi, c1, x, Wl_lo, Wl_hi, W_r, b, Wp, bp):
    M = 1000
    H = Wl_lo.shape[1]
    OW = Wp.shape[1]

    def body(lo_r, hi_r, c_r, x_r, wlo_r, whi_r, wr_r, b_r, wp_r, bp_r, o_r):
        inv = 1.0 / jnp.maximum(c_r[...], 1.0)
        acc = (_dot(lo_r[...] * inv, wlo_r[...])
               + _dot(hi_r[...] * inv, whi_r[...])
               + _dot(x_r[...], wr_r[...]) + b_r[...])
        h = jnp.maximum(acc, 0.0)
        o_r[...] = _dot(h, wp_r[...]) + bp_r[...]

    return pl.pallas_call(
        body,
        grid=(s_lo.shape[0] // M,),
        in_specs=[
            pl.BlockSpec((M, D), lambda i: (i, 0)),
            pl.BlockSpec((M, D), lambda i: (i, 0)),
            pl.BlockSpec((M, 1), lambda i: (i, 0)),
            pl.BlockSpec((M, D), lambda i: (i, 0)),
            pl.BlockSpec(Wl_lo.shape, lambda i: (0, 0)),
            pl.BlockSpec(Wl_hi.shape, lambda i: (0, 0)),
            pl.BlockSpec(W_r.shape, lambda i: (0, 0)),
            pl.BlockSpec((1, H), lambda i: (0, 0)),
            pl.BlockSpec(Wp.shape, lambda i: (0, 0)),
            pl.BlockSpec((1, OW), lambda i: (0, 0)),
        ],
        out_specs=pl.BlockSpec((M, OW), lambda i: (i, 0)),
        out_shape=jax.ShapeDtypeStruct((s_lo.shape[0], OW), jnp.float32),
    )(s_lo, s_hi, c1, x, Wl_lo, Wl_hi, W_r, b, Wp, bp)


def kernel(x_paper, x_author, edge_index_p2a, edge_index_a2p,
           W0_l, b0, W0_r, W1_l, b1, W1_r, Wp, bp):
    src0 = edge_index_p2a[0].astype(jnp.int32)
    dst0 = edge_index_p2a[1].astype(jnp.int32)
    src1 = edge_index_a2p[0].astype(jnp.int32)
    dst1 = edge_index_a2p[1].astype(jnp.int32)

    s0, c0 = _sc_aggregate(x_paper, src0, dst0)
    h_author = _tc_dense0(s0[:N_NODES], c0[:N_NODES, None], x_author,
                          W0_l, W0_r, b0[None, :])
    s1lo, c1 = _sc_aggregate(h_author[:, :D], src1, dst1)
    s1hi, _ = _sc_aggregate(h_author[:, D:], src1, dst1)
    out = _tc_dense1(s1lo[:N_NODES], s1hi[:N_NODES], c1[:N_NODES, None],
                     x_paper, W1_l[:D], W1_l[D:], W1_r, b1[None, :],
                     Wp, bp[None, :])
    return out


# speedup vs baseline: 3.0284x; 1.0993x over previous
import functools

import jax
import jax.numpy as jnp
from jax import lax
from jax.experimental import pallas as pl
from jax.experimental.pallas import tpu as pltpu
from jax.experimental.pallas import tpu_sc as plsc

N_NODES = 10000
E_TOTAL = 160000
NC = 2
NS = 16
L = 16
D = 256
NR = 10240
RNG = NR // (NC * NS)
MC = 1600
NM = E_TOTAL // MC
GC = 64

_MESH = plsc.VectorSubcoreMesh(
    core_axis_name="c", subcore_axis_name="s", num_cores=NC, num_subcores=NS
)


def _sc_aggregate(table, srcs, dsts):
    @functools.partial(
        pl.kernel,
        out_type=(
            jax.ShapeDtypeStruct((NR, D), jnp.float32),
            jax.ShapeDtypeStruct((NR,), jnp.float32),
        ),
        mesh=_MESH,
        compiler_params=pltpu.CompilerParams(needs_layout_passes=False),
        scratch_types=[
            pltpu.VMEM((MC,), jnp.int32),
            pltpu.VMEM((MC,), jnp.int32),
            pltpu.VMEM((MC,), jnp.int32),
            pltpu.VMEM((MC,), jnp.int32),
            pltpu.VMEM((GC, D), jnp.float32),
            pltpu.VMEM((RNG, D), jnp.float32),
            pltpu.VMEM((RNG,), jnp.float32),
            pltpu.SemaphoreType.DMA,
            pltpu.SemaphoreType.DMA,
        ],
    )
    def agg(table_h, src_h, dst_h, sum_o, cnt_o,
            smac0, dmac0, glist, dlist, rows, acc, lcnt, sem, sem0):
        c = lax.axis_index("c")
        s = lax.axis_index("s")
        wid = c * NS + s
        base = wid * RNG
        zeros16 = jnp.zeros((L,), jnp.float32)
        ones16 = jnp.ones((L,), jnp.float32)
        lanes = lax.iota(jnp.int32, L)
        lane0 = lanes == 0

        def _zacc(r, carry):
            def _zc(j, cc):
                acc[r, pl.ds(j * L, L)] = zeros16
                return cc
            return lax.fori_loop(0, D // L, _zc, carry)
        lax.fori_loop(0, RNG, _zacc, 0)

        def _zcnt(i, carry):
            lcnt[pl.ds(i * L, L)] = zeros16
            return carry
        lax.fori_loop(0, RNG // L, _zcnt, 0)

        zi16 = jnp.zeros((L,), jnp.int32)

        def _zg(i, carry):
            glist[pl.ds(i * L, L)] = zi16
            return carry
        lax.fori_loop(0, MC // L, _zg, 0)

        def _edge(dvec, ii, rp):
            dl_s = dvec[ii]
            rowv = jnp.full((L,), dl_s, jnp.int32)
            for jj in range(D // L):
                plsc.addupdate_scatter(acc, [rowv, lanes + jj * L],
                                       rows[rp, pl.ds(jj * L, L)])
            plsc.addupdate_scatter(lcnt, [rowv], ones16, mask=lane0)

        def _filter(smac, dmac):
            def _fchunk(j, off):
                d = dmac[pl.ds(j * L, L)]
                sv = smac[pl.ds(j * L, L)]
                dl = d - base
                owned = jnp.logical_and(dl >= 0, dl < RNG)
                incl = plsc.cumsum(owned.astype(jnp.int32))
                pos = off + incl - 1
                plsc.store_scatter(glist, [pos], sv, mask=owned)
                plsc.store_scatter(dlist, [pos], dl, mask=owned)
                return off + incl[L - 1]
            return lax.fori_loop(0, MC // L, _fchunk, jnp.int32(0))

        def _gather_acc(nown):
            nch = (nown + GC - 1) // GC

            def _chunk(cidx, carry):
                cb = cidx * GC
                pltpu.async_copy(table_h.at[glist.at[pl.ds(cb, GC)]],
                                 rows, sem).wait()
                ne = jnp.minimum(GC, nown - cb)

                def _sub(k, cc):
                    dvec = dlist[pl.ds(cb + k * L, L)]
                    for ii in range(L):
                        @pl.when(k * L + ii < ne)
                        def _():
                            _edge(dvec, ii, k * L + ii)
                    return cc
                lax.fori_loop(0, (ne + L - 1) // L, _sub, 0)
                return carry
            lax.fori_loop(0, nch, _chunk, 0)

        pltpu.async_copy(src_h.at[pl.ds(0, MC)], smac0, sem0)
        pltpu.async_copy(dst_h.at[pl.ds(0, MC)], dmac0, sem0)

        def _macro(m, carry):
            pltpu.make_async_copy(src_h.at[pl.ds(0, MC)], smac0, sem0).wait()
            pltpu.make_async_copy(dst_h.at[pl.ds(0, MC)], dmac0, sem0).wait()
            nown = _filter(smac0, dmac0)

            @pl.when(m + 1 < NM)
            def _():
                mb = (m + 1) * MC
                pltpu.async_copy(src_h.at[pl.ds(mb, MC)], smac0, sem0)
                pltpu.async_copy(dst_h.at[pl.ds(mb, MC)], dmac0, sem0)
            _gather_acc(nown)
            return carry
        lax.fori_loop(0, NM, _macro, 0)

        pltpu.sync_copy(acc, sum_o.at[pl.ds(base, RNG)])
        pltpu.sync_copy(lcnt, cnt_o.at[pl.ds(base, RNG)])

    return agg(table, srcs, dsts)


def _dot(a, b):
    return lax.dot_general(a, b, (((1,), (0,)), ((), ())),
                           preferred_element_type=jnp.float32)


def _tc_dense0(s0, c0, x, W_l, W_r, b):
    M = 1000
    H = W_l.shape[1]

    def body(s_r, c_r, x_r, wl_r, wr_r, b_r, o_r):
        mean = s_r[...] / jnp.maximum(c_r[...], 1.0)
        acc = _dot(mean, wl_r[...]) + _dot(x_r[...], wr_r[...]) + b_r[...]
        o_r[...] = jnp.maximum(acc, 0.0)

    return pl.pallas_call(
        body,
        grid=(s0.shape[0] // M,),
        in_specs=[
            pl.BlockSpec((M, D), lambda i: (i, 0)),
            pl.BlockSpec((M, 1), lambda i: (i, 0)),
            pl.BlockSpec((M, D), lambda i: (i, 0)),
            pl.BlockSpec(W_l.shape, lambda i: (0, 0)),
            pl.BlockSpec(W_r.shape, lambda i: (0, 0)),
            pl.BlockSpec((1, H), lambda i: (0, 0)),
        ],
        out_specs=pl.BlockSpec((M, H), lambda i: (i, 0)),
        out_shape=jax.ShapeDtypeStruct((s0.shape[0], H), jnp.float32),
    )(s0, c0, x, W_l, W_r, b)


def _tc_dense1(s_lo, s_hi, c1, x, Wl_lo, Wl_hi, W_r, b, Wp, bp):
    M = 1000
    H = Wl_lo.shape[1]
    OW = Wp.shape[1]

    def body(lo_r, hi_r, c_r, x_r, wlo_r, whi_r, wr_r, b_r, wp_r, bp_r, o_r):
        inv = 1.0 / jnp.maximum(c_r[...], 1.0)
        acc = (_dot(lo_r[...] * inv, wlo_r[...])
               + _dot(hi_r[...] * inv, whi_r[...])
               + _dot(x_r[...], wr_r[...]) + b_r[...])
        h = jnp.maximum(acc, 0.0)
        o_r[...] = _dot(h, wp_r[...]) + bp_r[...]

    return pl.pallas_call(
        body,
        grid=(s_lo.shape[0] // M,),
        in_specs=[
            pl.BlockSpec((M, D), lambda i: (i, 0)),
            pl.BlockSpec((M, D), lambda i: (i, 0)),
            pl.BlockSpec((M, 1), lambda i: (i, 0)),
            pl.BlockSpec((M, D), lambda i: (i, 0)),
            pl.BlockSpec(Wl_lo.shape, lambda i: (0, 0)),
            pl.BlockSpec(Wl_hi.shape, lambda i: (0, 0)),
            pl.BlockSpec(W_r.shape, lambda i: (0, 0)),
            pl.BlockSpec((1, H), lambda i: (0, 0)),
            pl.BlockSpec(Wp.shape, lambda i: (0, 0)),
            pl.BlockSpec((1, OW), lambda i: (0, 0)),
        ],
        out_specs=pl.BlockSpec((M, OW), lambda i: (i, 0)),
        out_shape=jax.ShapeDtypeStruct((s_lo.shape[0], OW), jnp.float32),
    )(s_lo, s_hi, c1, x, Wl_lo, Wl_hi, W_r, b, Wp, bp)


def kernel(x_paper, x_author, edge_index_p2a, edge_index_a2p,
           W0_l, b0, W0_r, W1_l, b1, W1_r, Wp, bp):
    src0 = edge_index_p2a[0].astype(jnp.int32)
    dst0 = edge_index_p2a[1].astype(jnp.int32)
    src1 = edge_index_a2p[0].astype(jnp.int32)
    dst1 = edge_index_a2p[1].astype(jnp.int32)

    s0, c0 = _sc_aggregate(x_paper, src0, dst0)
    h_author = _tc_dense0(s0[:N_NODES], c0[:N_NODES, None], x_author,
                          W0_l, W0_r, b0[None, :])
    s1lo, c1 = _sc_aggregate(h_author[:, :D], src1, dst1)
    s1hi, _ = _sc_aggregate(h_author[:, D:], src1, dst1)
    out = _tc_dense1(s1lo[:N_NODES], s1hi[:N_NODES], c1[:N_NODES, None],
                     x_paper, W1_l[:D], W1_l[D:], W1_r, b1[None, :],
                     Wp, bp[None, :])
    return out


# speedup vs baseline: 3.0798x; 1.0170x over previous
import functools

import jax
import jax.numpy as jnp
from jax import lax
from jax.experimental import pallas as pl
from jax.experimental.pallas import tpu as pltpu
from jax.experimental.pallas import tpu_sc as plsc

N_NODES = 10000
E_TOTAL = 160000
NC = 2
NS = 16
L = 16
D = 256
NR = 10240
RNG = NR // (NC * NS)
MC = 1600
NM = E_TOTAL // MC
GC = 64

_MESH = plsc.VectorSubcoreMesh(
    core_axis_name="c", subcore_axis_name="s", num_cores=NC, num_subcores=NS
)


def _sc_aggregate(table, srcs, dsts):
    @functools.partial(
        pl.kernel,
        out_type=(
            jax.ShapeDtypeStruct((NR, D), jnp.float32),
            jax.ShapeDtypeStruct((NR,), jnp.float32),
        ),
        mesh=_MESH,
        compiler_params=pltpu.CompilerParams(needs_layout_passes=False),
        scratch_types=[
            pltpu.VMEM((MC,), jnp.int32),
            pltpu.VMEM((MC,), jnp.int32),
            pltpu.VMEM((MC,), jnp.int32),
            pltpu.VMEM((MC,), jnp.int32),
            pltpu.VMEM((GC, D), jnp.float32),
            pltpu.VMEM((GC, D), jnp.float32),
            pltpu.VMEM((RNG, D), jnp.float32),
            pltpu.VMEM((RNG,), jnp.float32),
            pltpu.SemaphoreType.DMA,
            pltpu.SemaphoreType.DMA,
            pltpu.SemaphoreType.DMA,
        ],
    )
    def agg(table_h, src_h, dst_h, sum_o, cnt_o,
            smac0, dmac0, glist, dlist, rows0, rows1, acc, lcnt,
            semg0, semg1, sem0):
        c = lax.axis_index("c")
        s = lax.axis_index("s")
        wid = c * NS + s
        base = wid * RNG
        zeros16 = jnp.zeros((L,), jnp.float32)
        ones16 = jnp.ones((L,), jnp.float32)
        lanes = lax.iota(jnp.int32, L)
        lane0 = lanes == 0

        def _zacc(r, carry):
            def _zc(j, cc):
                acc[r, pl.ds(j * L, L)] = zeros16
                return cc
            return lax.fori_loop(0, D // L, _zc, carry)
        lax.fori_loop(0, RNG, _zacc, 0)

        def _zcnt(i, carry):
            lcnt[pl.ds(i * L, L)] = zeros16
            return carry
        lax.fori_loop(0, RNG // L, _zcnt, 0)

        zi16 = jnp.zeros((L,), jnp.int32)

        def _zg(i, carry):
            glist[pl.ds(i * L, L)] = zi16
            return carry
        lax.fori_loop(0, MC // L, _zg, 0)

        def _edge(rows, dvec, ii, rp):
            dl_s = dvec[ii]
            rowv = jnp.full((L,), dl_s, jnp.int32)
            for jj in range(D // L):
                plsc.addupdate_scatter(acc, [rowv, lanes + jj * L],
                                       rows[rp, pl.ds(jj * L, L)])
            plsc.addupdate_scatter(lcnt, [rowv], ones16, mask=lane0)

        def _filter(smac, dmac):
            def _fchunk(j, off):
                d = dmac[pl.ds(j * L, L)]
                sv = smac[pl.ds(j * L, L)]
                dl = d - base
                owned = jnp.logical_and(dl >= 0, dl < RNG)
                incl = plsc.cumsum(owned.astype(jnp.int32))
                pos = off + incl - 1
                plsc.store_scatter(glist, [pos], sv, mask=owned)
                plsc.store_scatter(dlist, [pos], dl, mask=owned)
                return off + incl[L - 1]
            return lax.fori_loop(0, MC // L, _fchunk, jnp.int32(0))

        def _issue_g(cidx, rows, semg):
            pltpu.async_copy(table_h.at[glist.at[pl.ds(cidx * GC, GC)]],
                             rows, semg)

        def _acc_chunk(rows, semg, cidx, nown):
            pltpu.make_async_copy(table_h.at[glist.at[pl.ds(0, GC)]],
                                  rows, semg).wait()
            cb = cidx * GC
            ne = jnp.minimum(GC, nown - cb)

            def _sub(k, cc):
                dvec = dlist[pl.ds(cb + k * L, L)]
                for ii in range(L):
                    @pl.when(k * L + ii < ne)
                    def _():
                        _edge(rows, dvec, ii, k * L + ii)
                return cc
            lax.fori_loop(0, (ne + L - 1) // L, _sub, 0)

        def _gather_acc(nown):
            nch = (nown + GC - 1) // GC

            @pl.when(nch > 0)
            def _():
                _issue_g(0, rows0, semg0)

            def _chunk(cidx, carry):
                even = lax.rem(cidx, 2) == 0

                @pl.when(jnp.logical_and(even, cidx + 1 < nch))
                def _():
                    _issue_g(cidx + 1, rows1, semg1)

                @pl.when(jnp.logical_and(~even, cidx + 1 < nch))
                def _():
                    _issue_g(cidx + 1, rows0, semg0)

                @pl.when(even)
                def _():
                    _acc_chunk(rows0, semg0, cidx, nown)

                @pl.when(~even)
                def _():
                    _acc_chunk(rows1, semg1, cidx, nown)
                return carry
            lax.fori_loop(0, nch, _chunk, 0)

        pltpu.async_copy(src_h.at[pl.ds(0, MC)], smac0, sem0)
        pltpu.async_copy(dst_h.at[pl.ds(0, MC)], dmac0, sem0)

        def _macro(m, carry):
            pltpu.make_async_copy(src_h.at[pl.ds(0, MC)], smac0, sem0).wait()
            pltpu.make_async_copy(dst_h.at[pl.ds(0, MC)], dmac0, sem0).wait()
            nown = _filter(smac0, dmac0)

            @pl.when(m + 1 < NM)
            def _():
                mb = (m + 1) * MC
                pltpu.async_copy(src_h.at[pl.ds(mb, MC)], smac0, sem0)
                pltpu.async_copy(dst_h.at[pl.ds(mb, MC)], dmac0, sem0)
            _gather_acc(nown)
            return carry
        lax.fori_loop(0, NM, _macro, 0)

        pltpu.sync_copy(acc, sum_o.at[pl.ds(base, RNG)])
        pltpu.sync_copy(lcnt, cnt_o.at[pl.ds(base, RNG)])

    return agg(table, srcs, dsts)


def _dot(a, b):
    return lax.dot_general(a, b, (((1,), (0,)), ((), ())),
                           preferred_element_type=jnp.float32)


def _tc_dense0(s0, c0, x, W_l, W_r, b):
    M = 1000
    H = W_l.shape[1]

    def body(s_r, c_r, x_r, wl_r, wr_r, b_r, o_r):
        mean = s_r[...] / jnp.maximum(c_r[...], 1.0)
        acc = _dot(mean, wl_r[...]) + _dot(x_r[...], wr_r[...]) + b_r[...]
        o_r[...] = jnp.maximum(acc, 0.0)

    return pl.pallas_call(
        body,
        grid=(s0.shape[0] // M,),
        in_specs=[
            pl.BlockSpec((M, D), lambda i: (i, 0)),
            pl.BlockSpec((M, 1), lambda i: (i, 0)),
            pl.BlockSpec((M, D), lambda i: (i, 0)),
            pl.BlockSpec(W_l.shape, lambda i: (0, 0)),
            pl.BlockSpec(W_r.shape, lambda i: (0, 0)),
            pl.BlockSpec((1, H), lambda i: (0, 0)),
        ],
        out_specs=pl.BlockSpec((M, H), lambda i: (i, 0)),
        out_shape=jax.ShapeDtypeStruct((s0.shape[0], H), jnp.float32),
    )(s0, c0, x, W_l, W_r, b)


def _tc_dense1(s_lo, s_hi, c1, x, Wl_lo, Wl_hi, W_r, b, Wp, bp):
    M = 1000
    H = Wl_lo.shape[1]
    OW = Wp.shape[1]

    def body(lo_r, hi_r, c_r, x_r, wlo_r, whi_r, wr_r, b_r, wp_r, bp_r, o_r):
        inv = 1.0 / jnp.maximum(c_r[...], 1.0)
        acc = (_dot(lo_r[...] * inv, wlo_r[...])
               + _dot(hi_r[...] * inv, whi_r[...])
               + _dot(x_r[...], wr_r[...]) + b_r[...])
        h = jnp.maximum(acc, 0.0)
        o_r[...] = _dot(h, wp_r[...]) + bp_r[...]

    return pl.pallas_call(
        body,
        grid=(s_lo.shape[0] // M,),
        in_specs=[
            pl.BlockSpec((M, D), lambda i: (i, 0)),
            pl.BlockSpec((M, D), lambda i: (i, 0)),
            pl.BlockSpec((M, 1), lambda i: (i, 0)),
            pl.BlockSpec((M, D), lambda i: (i, 0)),
            pl.BlockSpec(Wl_lo.shape, lambda i: (0, 0)),
            pl.BlockSpec(Wl_hi.shape, lambda i: (0, 0)),
            pl.BlockSpec(W_r.shape, lambda i: (0, 0)),
            pl.BlockSpec((1, H), lambda i: (0, 0)),
            pl.BlockSpec(Wp.shape, lambda i: (0, 0)),
            pl.BlockSpec((1, OW), lambda i: (0, 0)),
        ],
        out_specs=pl.BlockSpec((M, OW), lambda i: (i, 0)),
        out_shape=jax.ShapeDtypeStruct((s_lo.shape[0], OW), jnp.float32),
    )(s_lo, s_hi, c1, x, Wl_lo, Wl_hi, W_r, b, Wp, bp)


def kernel(x_paper, x_author, edge_index_p2a, edge_index_a2p,
           W0_l, b0, W0_r, W1_l, b1, W1_r, Wp, bp):
    src0 = edge_index_p2a[0].astype(jnp.int32)
    dst0 = edge_index_p2a[1].astype(jnp.int32)
    src1 = edge_index_a2p[0].astype(jnp.int32)
    dst1 = edge_index_a2p[1].astype(jnp.int32)

    s0, c0 = _sc_aggregate(x_paper, src0, dst0)
    h_author = _tc_dense0(s0[:N_NODES], c0[:N_NODES, None], x_author,
                          W0_l, W0_r, b0[None, :])
    s1lo, c1 = _sc_aggregate(h_author[:, :D], src1, dst1)
    s1hi, _ = _sc_aggregate(h_author[:, D:], src1, dst1)
    out = _tc_dense1(s1lo[:N_NODES], s1hi[:N_NODES], c1[:N_NODES, None],
                     x_paper, W1_l[:D], W1_l[D:], W1_r, b1[None, :],
                     Wp, bp[None, :])
    return out


# speedup vs baseline: 3.0839x; 1.0013x over previous
import functools

import jax
import jax.numpy as jnp
from jax import lax
from jax.experimental import pallas as pl
from jax.experimental.pallas import tpu as pltpu
from jax.experimental.pallas import tpu_sc as plsc

N_NODES = 10000
E_TOTAL = 160000
NC = 2
NS = 16
L = 16
D = 256
NR = 10240
RNG = NR // (NC * NS)
MC = 1600
NM = E_TOTAL // MC
GC = 64

_MESH = plsc.VectorSubcoreMesh(
    core_axis_name="c", subcore_axis_name="s", num_cores=NC, num_subcores=NS
)


def _sc_aggregate(table, srcs, dsts):
    @functools.partial(
        pl.kernel,
        out_type=(
            jax.ShapeDtypeStruct((NR, D), jnp.float32),
            jax.ShapeDtypeStruct((NR,), jnp.float32),
        ),
        mesh=_MESH,
        compiler_params=pltpu.CompilerParams(needs_layout_passes=False),
        scratch_types=[
            pltpu.VMEM((MC,), jnp.int32),
            pltpu.VMEM((MC,), jnp.int32),
            pltpu.VMEM((MC,), jnp.int32),
            pltpu.VMEM((MC,), jnp.int32),
            pltpu.VMEM((GC, D), jnp.float32),
            pltpu.VMEM((GC, D), jnp.float32),
            pltpu.VMEM((RNG, D), jnp.float32),
            pltpu.VMEM((RNG,), jnp.float32),
            pltpu.SemaphoreType.DMA,
            pltpu.SemaphoreType.DMA,
            pltpu.SemaphoreType.DMA,
        ],
    )
    def agg(table_h, src_h, dst_h, sum_o, cnt_o,
            smac0, dmac0, glist, dlist, rows0, rows1, acc, lcnt,
            semg0, semg1, sem0):
        c = lax.axis_index("c")
        s = lax.axis_index("s")
        wid = c * NS + s
        base = wid * RNG
        zeros16 = jnp.zeros((L,), jnp.float32)
        ones16 = jnp.ones((L,), jnp.float32)
        lanes = lax.iota(jnp.int32, L)
        lane0 = lanes == 0

        def _zacc(r, carry):
            def _zc(j, cc):
                acc[r, pl.ds(j * L, L)] = zeros16
                return cc
            return lax.fori_loop(0, D // L, _zc, carry)
        lax.fori_loop(0, RNG, _zacc, 0)

        def _zcnt(i, carry):
            lcnt[pl.ds(i * L, L)] = zeros16
            return carry
        lax.fori_loop(0, RNG // L, _zcnt, 0)

        zi16 = jnp.zeros((L,), jnp.int32)

        def _zg(i, carry):
            glist[pl.ds(i * L, L)] = zi16
            return carry
        lax.fori_loop(0, MC // L, _zg, 0)

        def _edge(rows, dvec, ii, rp):
            dl_s = dvec[ii]
            for jj in range(D // L):
                plsc.addupdate(acc.at[dl_s, pl.ds(jj * L, L)],
                               rows[rp, pl.ds(jj * L, L)])
            plsc.addupdate_scatter(lcnt, [jnp.full((L,), dl_s, jnp.int32)],
                                   ones16, mask=lane0)

        def _filter(smac, dmac):
            def _fchunk(j, off):
                d = dmac[pl.ds(j * L, L)]
                sv = smac[pl.ds(j * L, L)]
                dl = d - base
                owned = jnp.logical_and(dl >= 0, dl < RNG)
                incl = plsc.cumsum(owned.astype(jnp.int32))
                pos = off + incl - 1
                plsc.store_scatter(glist, [pos], sv, mask=owned)
                plsc.store_scatter(dlist, [pos], dl, mask=owned)
                return off + incl[L - 1]
            return lax.fori_loop(0, MC // L, _fchunk, jnp.int32(0))

        def _issue_g(cidx, rows, semg):
            pltpu.async_copy(table_h.at[glist.at[pl.ds(cidx * GC, GC)]],
                             rows, semg)

        def _acc_chunk(rows, semg, cidx, nown):
            pltpu.make_async_copy(table_h.at[glist.at[pl.ds(0, GC)]],
                                  rows, semg).wait()
            cb = cidx * GC
            ne = jnp.minimum(GC, nown - cb)

            def _sub(k, cc):
                dvec = dlist[pl.ds(cb + k * L, L)]
                for ii in range(L):
                    @pl.when(k * L + ii < ne)
                    def _():
                        _edge(rows, dvec, ii, k * L + ii)
                return cc
            lax.fori_loop(0, (ne + L - 1) // L, _sub, 0)

        def _gather_acc(nown):
            nch = (nown + GC - 1) // GC

            @pl.when(nch > 0)
            def _():
                _issue_g(0, rows0, semg0)

            def _chunk(cidx, carry):
                even = lax.rem(cidx, 2) == 0

                @pl.when(jnp.logical_and(even, cidx + 1 < nch))
                def _():
                    _issue_g(cidx + 1, rows1, semg1)

                @pl.when(jnp.logical_and(~even, cidx + 1 < nch))
                def _():
                    _issue_g(cidx + 1, rows0, semg0)

                @pl.when(even)
                def _():
                    _acc_chunk(rows0, semg0, cidx, nown)

                @pl.when(~even)
                def _():
                    _acc_chunk(rows1, semg1, cidx, nown)
                return carry
            lax.fori_loop(0, nch, _chunk, 0)

        pltpu.async_copy(src_h.at[pl.ds(0, MC)], smac0, sem0)
        pltpu.async_copy(dst_h.at[pl.ds(0, MC)], dmac0, sem0)

        def _macro(m, carry):
            pltpu.make_async_copy(src_h.at[pl.ds(0, MC)], smac0, sem0).wait()
            pltpu.make_async_copy(dst_h.at[pl.ds(0, MC)], dmac0, sem0).wait()
            nown = _filter(smac0, dmac0)

            @pl.when(m + 1 < NM)
            def _():
                mb = (m + 1) * MC
                pltpu.async_copy(src_h.at[pl.ds(mb, MC)], smac0, sem0)
                pltpu.async_copy(dst_h.at[pl.ds(mb, MC)], dmac0, sem0)
            _gather_acc(nown)
            return carry
        lax.fori_loop(0, NM, _macro, 0)

        pltpu.sync_copy(acc, sum_o.at[pl.ds(base, RNG)])
        pltpu.sync_copy(lcnt, cnt_o.at[pl.ds(base, RNG)])

    return agg(table, srcs, dsts)


def _dot(a, b):
    return lax.dot_general(a, b, (((1,), (0,)), ((), ())),
                           preferred_element_type=jnp.float32)


def _tc_dense0(s0, c0, x, W_l, W_r, b):
    M = 1000
    H = W_l.shape[1]

    def body(s_r, c_r, x_r, wl_r, wr_r, b_r, o_r):
        mean = s_r[...] / jnp.maximum(c_r[...], 1.0)
        acc = _dot(mean, wl_r[...]) + _dot(x_r[...], wr_r[...]) + b_r[...]
        o_r[...] = jnp.maximum(acc, 0.0)

    return pl.pallas_call(
        body,
        grid=(s0.shape[0] // M,),
        in_specs=[
            pl.BlockSpec((M, D), lambda i: (i, 0)),
            pl.BlockSpec((M, 1), lambda i: (i, 0)),
            pl.BlockSpec((M, D), lambda i: (i, 0)),
            pl.BlockSpec(W_l.shape, lambda i: (0, 0)),
            pl.BlockSpec(W_r.shape, lambda i: (0, 0)),
            pl.BlockSpec((1, H), lambda i: (0, 0)),
        ],
        out_specs=pl.BlockSpec((M, H), lambda i: (i, 0)),
        out_shape=jax.ShapeDtypeStruct((s0.shape[0], H), jnp.float32),
    )(s0, c0, x, W_l, W_r, b)


def _tc_dense1(s_lo, s_hi, c1, x, Wl_lo, Wl_hi, W_r, b, Wp, bp):
    M = 1000
    H = Wl_lo.shape[1]
    OW = Wp.shape[1]

    def body(lo_r, hi_r, c_r, x_r, wlo_r, whi_r, wr_r, b_r, wp_r, bp_r, o_r):
        inv = 1.0 / jnp.maximum(c_r[...], 1.0)
        acc = (_dot(lo_r[...] * inv, wlo_r[...])
               + _dot(hi_r[...] * inv, whi_r[...])
               + _dot(x_r[...], wr_r[...]) + b_r[...])
        h = jnp.maximum(acc, 0.0)
        o_r[...] = _dot(h, wp_r[...]) + bp_r[...]

    return pl.pallas_call(
        body,
        grid=(s_lo.shape[0] // M,),
        in_specs=[
            pl.BlockSpec((M, D), lambda i: (i, 0)),
            pl.BlockSpec((M, D), lambda i: (i, 0)),
            pl.BlockSpec((M, 1), lambda i: (i, 0)),
            pl.BlockSpec((M, D), lambda i: (i, 0)),
            pl.BlockSpec(Wl_lo.shape, lambda i: (0, 0)),
            pl.BlockSpec(Wl_hi.shape, lambda i: (0, 0)),
            pl.BlockSpec(W_r.shape, lambda i: (0, 0)),
            pl.BlockSpec((1, H), lambda i: (0, 0)),
            pl.BlockSpec(Wp.shape, lambda i: (0, 0)),
            pl.BlockSpec((1, OW), lambda i: (0, 0)),
        ],
        out_specs=pl.BlockSpec((M, OW), lambda i: (i, 0)),
        out_shape=jax.ShapeDtypeStruct((s_lo.shape[0], OW), jnp.float32),
    )(s_lo, s_hi, c1, x, Wl_lo, Wl_hi, W_r, b, Wp, bp)


def kernel(x_paper, x_author, edge_index_p2a, edge_index_a2p,
           W0_l, b0, W0_r, W1_l, b1, W1_r, Wp, bp):
    src0 = edge_index_p2a[0].astype(jnp.int32)
    dst0 = edge_index_p2a[1].astype(jnp.int32)
    src1 = edge_index_a2p[0].astype(jnp.int32)
    dst1 = edge_index_a2p[1].astype(jnp.int32)

    s0, c0 = _sc_aggregate(x_paper, src0, dst0)
    h_author = _tc_dense0(s0[:N_NODES], c0[:N_NODES, None], x_author,
                          W0_l, W0_r, b0[None, :])
    s1lo, c1 = _sc_aggregate(h_author[:, :D], src1, dst1)
    s1hi, _ = _sc_aggregate(h_author[:, D:], src1, dst1)
    out = _tc_dense1(s1lo[:N_NODES], s1hi[:N_NODES], c1[:N_NODES, None],
                     x_paper, W1_l[:D], W1_l[D:], W1_r, b1[None, :],
                     Wp, bp[None, :])
    return out
